# baseline (speedup 1.0000x reference)
import functools

import jax
import jax.numpy as jnp
from jax import lax
from jax.experimental import pallas as pl
from jax.experimental.pallas import tpu as pltpu
from jax.experimental.pallas import tpu_sc as plsc

_NW = 32
_C = 200
_NBUF = 8
_SPLITS = ((0, 128), (128, 72))
_LANES = 16


def _emb_body(idx_hbm, pos_hbm, tok_hbm, out_hbm, idx_v, gbuf_v, pos_v,
              sem_g, sem_s, *, rows_per_w, seq_len, embed):
    nc = 2
    wid = lax.axis_index("s") * nc + lax.axis_index("c")
    base = wid * rows_per_w
    n_chunks = rows_per_w // _C
    quarter = embed // _LANES

    pltpu.sync_copy(pos_hbm, pos_v)

    def gather_copy(off, sz, b):
        return pltpu.make_async_copy(
            tok_hbm.at[idx_v.at[b, pl.ds(off, sz)]],
            gbuf_v.at[b, pl.ds(off, sz), :],
            sem_g.at[b],
        )

    def store_copy(rbase, b):
        return pltpu.make_async_copy(
            gbuf_v.at[b],
            out_hbm.at[pl.ds(rbase, _C), pl.ds(0, embed)],
            sem_s.at[b],
        )

    def fire(g, b, first):
        rbase = base + g * _C
        if not first:
            store_copy(rbase - _NBUF * _C, b).wait()
        pltpu.sync_copy(idx_hbm.at[pl.ds(rbase, _C)], idx_v.at[b])
        for off, sz in _SPLITS:
            gather_copy(off, sz, b).start()

    def proc(g, b):
        for off, sz in _SPLITS:
            gather_copy(off, sz, b).wait()

        @plsc.parallel_loop(0, seq_len, 1, unroll=4)
        def _(r):
            for u in range(quarter):
                sl = pl.ds(u * _LANES, _LANES)
                psl = pl.ds(r * embed + u * _LANES, _LANES)
                gbuf_v[b, r, sl] = gbuf_v[b, r, sl] + pos_v[psl]

        store_copy(base + g * _C, b).start()

    for b in range(_NBUF):
        fire(b, b, first=True)

    def loop_body(it, carry):
        g0 = it * _NBUF
        for b in range(_NBUF):
            proc(g0 + b, b)
        for b in range(_NBUF):
            fire(g0 + _NBUF + b, b, first=False)
        return carry

    lax.fori_loop(0, n_chunks // _NBUF - 1, loop_body, 0)

    g_last = n_chunks - _NBUF
    for b in range(_NBUF):
        proc(g_last + b, b)
    for b in range(_NBUF):
        store_copy(base + (g_last + b) * _C, b).wait()


def kernel(x, token_table, pos_table):
    batch, seq_len = x.shape
    _, embed = token_table.shape
    n = batch * seq_len
    rows_per_w = n // _NW

    idx_flat = x.reshape(n).astype(jnp.int32)
    pos_flat = pos_table.reshape(seq_len * embed)

    mesh = plsc.VectorSubcoreMesh(core_axis_name="c", subcore_axis_name="s")
    body = functools.partial(
        _emb_body, rows_per_w=rows_per_w, seq_len=seq_len, embed=embed
    )
    out = pl.kernel(
        body,
        out_type=jax.ShapeDtypeStruct((n, 2 * embed), jnp.float32),
        mesh=mesh,
        scratch_types=[
            pltpu.VMEM((_NBUF, _C), jnp.int32),
            pltpu.VMEM((_NBUF, _C, embed), jnp.float32),
            pltpu.VMEM((seq_len * embed,), jnp.float32),
            pltpu.SemaphoreType.DMA((_NBUF,)),
            pltpu.SemaphoreType.DMA((_NBUF,)),
        ],
        compiler_params=pltpu.CompilerParams(use_tc_tiling_on_sc=False),
    )(idx_flat, pos_flat, token_table)
    return out[:, :embed].reshape(batch, seq_len, embed)


# speedup vs baseline: 1.0490x; 1.0490x over previous
import functools

import jax
import jax.numpy as jnp
from jax import lax
from jax.experimental import pallas as pl
from jax.experimental.pallas import tpu as pltpu
from jax.experimental.pallas import tpu_sc as plsc

_NW = 32
_C = 800
_NBUF = 2
_SPLITS = ((0, 128), (128, 128), (256, 128), (384, 128),
           (512, 128), (640, 128), (768, 32))
_LANES = 16


def _emb_body(idx_hbm, pos_hbm, tok_hbm, out_hbm, idx_v, gbuf_v, pos_v,
              sem_g, sem_s, *, rows_per_w, seq_len, embed):
    nc = 2
    wid = lax.axis_index("s") * nc + lax.axis_index("c")
    base = wid * rows_per_w
    n_chunks = rows_per_w // _C
    quarter = embed // _LANES

    pltpu.sync_copy(pos_hbm, pos_v)

    def gather_copy(off, sz, b):
        return pltpu.make_async_copy(
            tok_hbm.at[idx_v.at[b, pl.ds(off, sz)]],
            gbuf_v.at[b, pl.ds(off, sz), :],
            sem_g.at[b],
        )

    def store_copy(rbase, b):
        return pltpu.make_async_copy(
            gbuf_v.at[b],
            out_hbm.at[pl.ds(rbase, _C), pl.ds(0, embed)],
            sem_s.at[b],
        )

    def fire(g, b, first):
        rbase = base + g * _C
        if not first:
            store_copy(rbase - _NBUF * _C, b).wait()
        pltpu.sync_copy(idx_hbm.at[pl.ds(rbase, _C)], idx_v.at[b])
        for off, sz in _SPLITS:
            gather_copy(off, sz, b).start()

    def proc(g, b):
        for off, sz in _SPLITS:
            gather_copy(off, sz, b).wait()

        @plsc.parallel_loop(0, seq_len, 1, unroll=2)
        def _(r):
            for dr in range(0, _C, seq_len):
                for u in range(quarter):
                    sl = pl.ds(u * _LANES, _LANES)
                    psl = pl.ds(r * embed + u * _LANES, _LANES)
                    gbuf_v[b, r + dr, sl] = gbuf_v[b, r + dr, sl] + pos_v[psl]

        store_copy(base + g * _C, b).start()

    for b in range(_NBUF):
        fire(b, b, first=True)

    def loop_body(it, carry):
        g0 = it * _NBUF
        for b in range(_NBUF):
            proc(g0 + b, b)
        for b in range(_NBUF):
            fire(g0 + _NBUF + b, b, first=False)
        return carry

    lax.fori_loop(0, n_chunks // _NBUF - 1, loop_body, 0)

    g_last = n_chunks - _NBUF
    for b in range(_NBUF):
        proc(g_last + b, b)
    for b in range(_NBUF):
        store_copy(base + (g_last + b) * _C, b).wait()


def kernel(x, token_table, pos_table):
    batch, seq_len = x.shape
    _, embed = token_table.shape
    n = batch * seq_len
    rows_per_w = n // _NW

    idx_flat = x.reshape(n).astype(jnp.int32)
    pos_flat = pos_table.reshape(seq_len * embed)

    mesh = plsc.VectorSubcoreMesh(core_axis_name="c", subcore_axis_name="s")
    body = functools.partial(
        _emb_body, rows_per_w=rows_per_w, seq_len=seq_len, embed=embed
    )
    out = pl.kernel(
        body,
        out_type=jax.ShapeDtypeStruct((n, 2 * embed), jnp.float32),
        mesh=mesh,
        scratch_types=[
            pltpu.VMEM((_NBUF, _C), jnp.int32),
            pltpu.VMEM((_NBUF, _C, embed), jnp.float32),
            pltpu.VMEM((seq_len * embed,), jnp.float32),
            pltpu.SemaphoreType.DMA((_NBUF,)),
            pltpu.SemaphoreType.DMA((_NBUF,)),
        ],
        compiler_params=pltpu.CompilerParams(use_tc_tiling_on_sc=False),
    )(idx_flat, pos_flat, token_table)
    return out[:, :embed].reshape(batch, seq_len, embed)


# speedup vs baseline: 1.0700x; 1.0200x over previous
import functools

import jax
import jax.numpy as jnp
from jax import lax
from jax.experimental import pallas as pl
from jax.experimental.pallas import tpu as pltpu
from jax.experimental.pallas import tpu_sc as plsc

_NW = 32
_C = 400
_NBUF = 4
_SPLITS = ((0, 128), (128, 128), (256, 128), (384, 16))
_LANES = 16


def _emb_body(idx_hbm, pos_hbm, tok_hbm, out_hbm, idx_v, gbuf_v, pos_v,
              sem_g, sem_s, *, rows_per_w, seq_len, embed):
    nc = 2
    wid = lax.axis_index("s") * nc + lax.axis_index("c")
    base = wid * rows_per_w
    n_chunks = rows_per_w // _C
    quarter = embed // _LANES

    pltpu.sync_copy(pos_hbm, pos_v)

    def gather_copy(off, sz, b):
        return pltpu.make_async_copy(
            tok_hbm.at[idx_v.at[b, pl.ds(off, sz)]],
            gbuf_v.at[b, pl.ds(off, sz), :],
            sem_g.at[b],
        )

    def store_copy(rbase, b):
        return pltpu.make_async_copy(
            gbuf_v.at[b],
            out_hbm.at[pl.ds(rbase, _C), pl.ds(0, embed)],
            sem_s.at[b],
        )

    def fire(g, b, first):
        rbase = base + g * _C
        if not first:
            store_copy(rbase - _NBUF * _C, b).wait()
        pltpu.sync_copy(idx_hbm.at[pl.ds(rbase, _C)], idx_v.at[b])
        for off, sz in _SPLITS:
            gather_copy(off, sz, b).start()

    def proc(g, b):
        for off, sz in _SPLITS:
            gather_copy(off, sz, b).wait()

        @plsc.parallel_loop(0, seq_len, 1, unroll=2)
        def _(r):
            for dr in (0, seq_len):
                for u in range(quarter):
                    sl = pl.ds(u * _LANES, _LANES)
                    psl = pl.ds(r * embed + u * _LANES, _LANES)
                    gbuf_v[b, r + dr, sl] = gbuf_v[b, r + dr, sl] + pos_v[psl]

        store_copy(base + g * _C, b).start()

    for b in range(_NBUF):
        fire(b, b, first=True)

    def loop_body(it, carry):
        g0 = it * _NBUF
        for b in range(_NBUF):
            proc(g0 + b, b)
        for b in range(_NBUF):
            fire(g0 + _NBUF + b, b, first=False)
        return carry

    lax.fori_loop(0, n_chunks // _NBUF - 1, loop_body, 0)

    g_last = n_chunks - _NBUF
    for b in range(_NBUF):
        proc(g_last + b, b)
    for b in range(_NBUF):
        store_copy(base + (g_last + b) * _C, b).wait()


def kernel(x, token_table, pos_table):
    batch, seq_len = x.shape
    _, embed = token_table.shape
    n = batch * seq_len
    rows_per_w = n // _NW

    idx_flat = x.reshape(n).astype(jnp.int32)
    pos_flat = pos_table.reshape(seq_len * embed)

    mesh = plsc.VectorSubcoreMesh(core_axis_name="c", subcore_axis_name="s")
    body = functools.partial(
        _emb_body, rows_per_w=rows_per_w, seq_len=seq_len, embed=embed
    )
    out = pl.kernel(
        body,
        out_type=jax.ShapeDtypeStruct((n, 2 * embed), jnp.float32),
        mesh=mesh,
        scratch_types=[
            pltpu.VMEM((_NBUF, _C), jnp.int32),
            pltpu.VMEM((_NBUF, _C, embed), jnp.float32),
            pltpu.VMEM((seq_len * embed,), jnp.float32),
            pltpu.SemaphoreType.DMA((_NBUF,)),
            pltpu.SemaphoreType.DMA((_NBUF,)),
        ],
        compiler_params=pltpu.CompilerParams(use_tc_tiling_on_sc=False),
    )(idx_flat, pos_flat, token_table)
    return out[:, :embed].reshape(batch, seq_len, embed)
